# Initial kernel scaffold; baseline (speedup 1.0000x reference)
import jax
import jax.numpy as jnp
from jax.experimental import pallas as pl


def kernel(sigs, frac_applicable_embed, bool_true_embed, bool_false_embed, frac_tf_embed):
    raise NotImplementedError("write your pallas kernel here")


# speedup vs baseline: 16.2668x; 16.2668x over previous
import functools

import numpy as np

import jax
import jax.numpy as jnp
from jax import lax
from jax.experimental import pallas as pl
from jax.experimental.pallas import tpu as pltpu
from jax.experimental.pallas import tpu_sc as plsc

_B = 4096
_L = 50
_NW = 32
_ROWS_W = _B // _NW
_IN_W = _ROWS_W * _L * 4
_OUT_W = _ROWS_W * _L * 8
_PAIRS = _ROWS_W // 2
_CHUNKS = (2 * _L * 4) // 16


def _consts():
    fid = np.arange(16) % 4
    scale = np.where((fid == 0) | (fid == 3), 10.0, 1.0).astype(np.float32)
    base = np.array([0, 22, 26, 30], dtype=np.int32)[fid]
    luts = np.concatenate([base, base + 1]).astype(np.int32)
    jj = np.arange(2 * _L * 4)
    r2, rem = jj // 200, jj % 200
    d0 = r2 * 400 + (rem % 4) * 100 + 2 * (rem // 4)
    dst = np.concatenate([d0, d0 + 1]).astype(np.int32)
    return jnp.asarray(scale), jnp.asarray(luts), jnp.asarray(dst)


_mesh = plsc.VectorSubcoreMesh(core_axis_name="c", subcore_axis_name="s")


@functools.partial(
    pl.kernel,
    out_type=jax.ShapeDtypeStruct((_B * _L * 8,), jnp.float32),
    mesh=_mesh,
    compiler_params=pltpu.CompilerParams(needs_layout_passes=False),
    scratch_types=[
        pltpu.VMEM((_IN_W,), jnp.float32),
        pltpu.VMEM((_OUT_W,), jnp.float32),
        pltpu.VMEM((64,), jnp.float32),
        pltpu.VMEM((16,), jnp.float32),
        pltpu.VMEM((32,), jnp.int32),
        pltpu.VMEM((2 * 2 * _L * 4,), jnp.int32),
    ],
)
def _sc_lookup(sigs_hbm, tab_hbm, scale_hbm, luts_hbm, dst_hbm, out_hbm,
               sigs_v, out_v, tab_v, scale_v, luts_v, dst_v):
    wid = lax.axis_index("s") * 2 + lax.axis_index("c")
    pltpu.sync_copy(sigs_hbm.at[pl.ds(wid * _IN_W, _IN_W)], sigs_v)
    pltpu.sync_copy(tab_hbm, tab_v)
    pltpu.sync_copy(scale_hbm, scale_v)
    pltpu.sync_copy(luts_hbm, luts_v)
    pltpu.sync_copy(dst_hbm, dst_v)

    scale = scale_v[...]
    lut0 = luts_v[pl.ds(0, 16)]
    lut1 = luts_v[pl.ds(16, 16)]

    def k_body(k, _):
        k16 = k * 16
        d0 = dst_v[pl.ds(k16, 16)]
        d1 = dst_v[pl.ds(400 + k16, 16)]

        def p_body(p, _):
            s = sigs_v[pl.ds(p * 400 + k16, 16)]
            t2 = (s * scale).astype(jnp.int32)
            t2 = t2 + t2
            v0 = plsc.load_gather(tab_v, [lut0 + t2])
            v1 = plsc.load_gather(tab_v, [lut1 + t2])
            ob = out_v.at[pl.ds(p * 800, 800)]
            plsc.store_scatter(ob, [d0], v0)
            plsc.store_scatter(ob, [d1], v1)
            return 0

        lax.fori_loop(0, _PAIRS, p_body, 0)
        return 0

    lax.fori_loop(0, _CHUNKS, k_body, 0)

    pltpu.sync_copy(out_v, out_hbm.at[pl.ds(wid * _OUT_W, _OUT_W)])


def kernel(sigs, frac_applicable_embed, bool_true_embed, bool_false_embed, frac_tf_embed):
    B, L, _ = sigs.shape
    tab = jnp.concatenate([
        frac_applicable_embed.reshape(-1),
        bool_true_embed.reshape(-1),
        bool_false_embed.reshape(-1),
        frac_tf_embed.reshape(-1),
    ])
    tab = jnp.pad(tab, (0, 64 - tab.shape[0]))
    scale, luts, dst = _consts()
    out = _sc_lookup(sigs.reshape(B * L * 4), tab, scale, luts, dst)
    return out.reshape(B, L * 8)


# speedup vs baseline: 18.3513x; 1.1281x over previous
import functools

import numpy as np

import jax
import jax.numpy as jnp
from jax import lax
from jax.experimental import pallas as pl
from jax.experimental.pallas import tpu as pltpu
from jax.experimental.pallas import tpu_sc as plsc

_B = 4096
_L = 50
_NW = 32
_ROWS_W = _B // _NW
_IN_W = _ROWS_W * _L * 4
_OUT_W = _ROWS_W * _L * 8
_PAIRS = _ROWS_W // 2
_CHUNKS = (2 * _L * 4) // 16


def _consts():
    fid = np.arange(16) % 4
    scale = np.where((fid == 0) | (fid == 3), 10.0, 1.0).astype(np.float32)
    base = np.array([0, 22, 26, 30], dtype=np.int32)[fid]
    luts = np.concatenate([base, base + 1]).astype(np.int32)
    jj = np.arange(2 * _L * 4)
    r2, rem = jj // 200, jj % 200
    d0 = r2 * 400 + (rem % 4) * 100 + 2 * (rem // 4)
    dst = np.concatenate([d0, d0 + 1]).astype(np.int32)
    return jnp.asarray(scale), jnp.asarray(luts), jnp.asarray(dst)


_mesh = plsc.VectorSubcoreMesh(core_axis_name="c", subcore_axis_name="s")


@functools.partial(
    pl.kernel,
    out_type=jax.ShapeDtypeStruct((_B * _L * 8,), jnp.float32),
    mesh=_mesh,
    compiler_params=pltpu.CompilerParams(needs_layout_passes=False),
    scratch_types=[
        pltpu.VMEM((_IN_W,), jnp.float32),
        pltpu.VMEM((_OUT_W,), jnp.float32),
        pltpu.VMEM((64,), jnp.float32),
        pltpu.VMEM((16,), jnp.float32),
        pltpu.VMEM((32,), jnp.int32),
        pltpu.VMEM((2 * 2 * _L * 4,), jnp.int32),
    ],
)
def _sc_lookup(sigs_hbm, tab_hbm, scale_hbm, luts_hbm, dst_hbm, out_hbm,
               sigs_v, out_v, tab_v, scale_v, luts_v, dst_v):
    wid = lax.axis_index("s") * 2 + lax.axis_index("c")
    pltpu.sync_copy(sigs_hbm.at[pl.ds(wid * _IN_W, _IN_W)], sigs_v)
    pltpu.sync_copy(tab_hbm, tab_v)
    pltpu.sync_copy(scale_hbm, scale_v)
    pltpu.sync_copy(luts_hbm, luts_v)
    pltpu.sync_copy(dst_hbm, dst_v)

    scale = scale_v[...]
    lut0 = luts_v[pl.ds(0, 16)]
    lut1 = luts_v[pl.ds(16, 16)]

    def k_body(k, _):
        k16 = k * 16
        d0 = dst_v[pl.ds(k16, 16)]
        d1 = dst_v[pl.ds(400 + k16, 16)]

        @plsc.parallel_loop(0, _PAIRS, 1, unroll=8)
        def p_body(p):
            s = sigs_v[pl.ds(p * 400 + k16, 16)]
            t2 = (s * scale).astype(jnp.int32)
            t2 = t2 + t2
            v0 = plsc.load_gather(tab_v, [lut0 + t2])
            v1 = plsc.load_gather(tab_v, [lut1 + t2])
            ob = out_v.at[pl.ds(p * 800, 800)]
            plsc.store_scatter(ob, [d0], v0)
            plsc.store_scatter(ob, [d1], v1)

        return 0

    lax.fori_loop(0, _CHUNKS, k_body, 0)

    pltpu.sync_copy(out_v, out_hbm.at[pl.ds(wid * _OUT_W, _OUT_W)])


def kernel(sigs, frac_applicable_embed, bool_true_embed, bool_false_embed, frac_tf_embed):
    B, L, _ = sigs.shape
    tab = jnp.concatenate([
        frac_applicable_embed.reshape(-1),
        bool_true_embed.reshape(-1),
        bool_false_embed.reshape(-1),
        frac_tf_embed.reshape(-1),
    ])
    tab = jnp.pad(tab, (0, 64 - tab.shape[0]))
    scale, luts, dst = _consts()
    out = _sc_lookup(sigs.reshape(B * L * 4), tab, scale, luts, dst)
    return out.reshape(B, L * 8)


# speedup vs baseline: 19.2556x; 1.0493x over previous
import functools

import numpy as np

import jax
import jax.numpy as jnp
from jax import lax
from jax.experimental import pallas as pl
from jax.experimental.pallas import tpu as pltpu
from jax.experimental.pallas import tpu_sc as plsc

_B = 4096
_L = 50
_NW = 32
_ROWS_W = _B // _NW
_IN_W = _ROWS_W * _L * 4
_PAIRS = _ROWS_W // 2
_CHUNKS = (2 * _L * 4) // 16


def _consts():
    fid = np.arange(16) % 4
    scale = np.where((fid == 0) | (fid == 3), 10.0, 1.0).astype(np.float32)
    base = np.array([0, 22, 26, 30], dtype=np.int32)[fid]
    luts = np.concatenate([base, base + 1]).astype(np.int32)
    jj = np.arange(2 * _L * 4)
    r2, rem = jj // 200, jj % 200
    c0 = (rem % 4) * 100 + 2 * (rem // 4)
    dstr = r2.astype(np.int32)
    dstc = np.concatenate([c0, c0 + 1]).astype(np.int32)
    return jnp.asarray(scale), jnp.asarray(luts), jnp.asarray(dstr), jnp.asarray(dstc)


_mesh = plsc.VectorSubcoreMesh(core_axis_name="c", subcore_axis_name="s")


@functools.partial(
    pl.kernel,
    out_type=jax.ShapeDtypeStruct((_B, _L * 8), jnp.float32),
    mesh=_mesh,
    compiler_params=pltpu.CompilerParams(needs_layout_passes=False),
    scratch_types=[
        pltpu.VMEM((_IN_W,), jnp.float32),
        pltpu.VMEM((_ROWS_W, _L * 8), jnp.float32),
        pltpu.VMEM((64,), jnp.float32),
        pltpu.VMEM((16,), jnp.float32),
        pltpu.VMEM((32,), jnp.int32),
        pltpu.VMEM((2 * _L * 4,), jnp.int32),
        pltpu.VMEM((2 * 2 * _L * 4,), jnp.int32),
    ],
)
def _sc_lookup(sigs_hbm, tab_hbm, scale_hbm, luts_hbm, dstr_hbm, dstc_hbm, out_hbm,
               sigs_v, out_v, tab_v, scale_v, luts_v, dstr_v, dstc_v):
    wid = lax.axis_index("s") * 2 + lax.axis_index("c")
    pltpu.sync_copy(sigs_hbm.at[pl.ds(wid * _IN_W, _IN_W)], sigs_v)
    pltpu.sync_copy(tab_hbm, tab_v)
    pltpu.sync_copy(scale_hbm, scale_v)
    pltpu.sync_copy(luts_hbm, luts_v)
    pltpu.sync_copy(dstr_hbm, dstr_v)
    pltpu.sync_copy(dstc_hbm, dstc_v)

    scale = scale_v[...]
    lut0 = luts_v[pl.ds(0, 16)]
    lut1 = luts_v[pl.ds(16, 16)]

    def k_body(k, _):
        k16 = k * 16
        dr = dstr_v[pl.ds(k16, 16)]
        c0 = dstc_v[pl.ds(k16, 16)]
        c1 = dstc_v[pl.ds(400 + k16, 16)]

        @plsc.parallel_loop(0, _PAIRS, 1, unroll=8)
        def p_body(p):
            s = sigs_v[pl.ds(p * 400 + k16, 16)]
            t2 = (s * scale).astype(jnp.int32)
            t2 = t2 + t2
            v0 = plsc.load_gather(tab_v, [lut0 + t2])
            v1 = plsc.load_gather(tab_v, [lut1 + t2])
            drp = dr + p * 2
            plsc.store_scatter(out_v, [drp, c0], v0)
            plsc.store_scatter(out_v, [drp, c1], v1)

        return 0

    lax.fori_loop(0, _CHUNKS, k_body, 0)

    pltpu.sync_copy(out_v, out_hbm.at[pl.ds(wid * _ROWS_W, _ROWS_W)])


def kernel(sigs, frac_applicable_embed, bool_true_embed, bool_false_embed, frac_tf_embed):
    B, L, _ = sigs.shape
    tab = jnp.concatenate([
        frac_applicable_embed.reshape(-1),
        bool_true_embed.reshape(-1),
        bool_false_embed.reshape(-1),
        frac_tf_embed.reshape(-1),
    ])
    tab = jnp.pad(tab, (0, 64 - tab.shape[0]))
    scale, luts, dstr, dstc = _consts()
    return _sc_lookup(sigs.reshape(B * L * 4), tab, scale, luts, dstr, dstc)


# speedup vs baseline: 43.9715x; 2.2836x over previous
import functools

import numpy as np

import jax
import jax.numpy as jnp
from jax import lax
from jax.experimental import pallas as pl
from jax.experimental.pallas import tpu as pltpu
from jax.experimental.pallas import tpu_sc as plsc

_B = 4096
_L = 50
_NW = 32
_ROWS_W = _B // _NW
_HROWS = _ROWS_W // 2
_OCTS = _HROWS // 8
_CHUNKS = (8 * _L) // 16

_BASES = (0, 22, 26, 30)
_SCALED = (True, False, False, True)


def _consts():
    jj = np.arange(8 * _L)
    srcr = (jj // _L).astype(np.int32)
    srcl = (jj % _L).astype(np.int32)
    dstc = (2 * (jj % _L)).astype(np.int32)
    return jnp.asarray(srcr), jnp.asarray(srcl), jnp.asarray(dstc)


_mesh = plsc.VectorSubcoreMesh(core_axis_name="c", subcore_axis_name="s")


@functools.partial(
    pl.kernel,
    out_type=jax.ShapeDtypeStruct((_B, _L * 8), jnp.float32),
    name="lambda_signature_lookup",
    mesh=_mesh,
    compiler_params=pltpu.CompilerParams(needs_layout_passes=False),
    scratch_types=[
        [pltpu.VMEM((_HROWS, _L), jnp.float32) for _ in range(4)],
        pltpu.VMEM((_HROWS, _L * 8), jnp.float32),
        pltpu.VMEM((64,), jnp.float32),
        pltpu.VMEM((3 * 8 * _L,), jnp.int32),
    ],
)
def _sc_lookup(s0_hbm, s1_hbm, s2_hbm, s3_hbm, tab_hbm, idx_hbm, out_hbm,
               sig_vs, out_v, tab_v, idx_v):
    wid = lax.axis_index("s") * 2 + lax.axis_index("c")
    base_row = wid * _ROWS_W
    pltpu.sync_copy(tab_hbm, tab_v)
    pltpu.sync_copy(idx_hbm, idx_v)
    s_hbm = (s0_hbm, s1_hbm, s2_hbm, s3_hbm)

    for h in range(2):
        rows = pl.ds(base_row + h * _HROWS, _HROWS)
        for f in range(4):
            pltpu.sync_copy(s_hbm[f].at[rows], sig_vs[f])

        for f in range(4):
            sv = sig_vs[f]
            base = _BASES[f]

            def k_body(k, _, sv=sv, base=base, scaled=_SCALED[f], fcol=f * 100):
                k16 = k * 16
                sr = idx_v[pl.ds(k16, 16)]
                sl = idx_v[pl.ds(400 + k16, 16)]
                c0 = idx_v[pl.ds(800 + k16, 16)] + fcol

                @plsc.parallel_loop(0, _OCTS, 1, unroll=8)
                def o_body(o):
                    ro = sr + o * 8
                    s = plsc.load_gather(sv, [ro, sl])
                    if scaled:
                        s = s * jnp.float32(10.0)
                    t = s.astype(jnp.int32)
                    idx = t + t + base
                    v0 = plsc.load_gather(tab_v, [idx])
                    v1 = plsc.load_gather(tab_v, [idx + 1])
                    plsc.store_scatter(out_v, [ro, c0], v0)
                    plsc.store_scatter(out_v, [ro, c0 + 1], v1)

                return 0

            lax.fori_loop(0, _CHUNKS, k_body, 0)

        pltpu.sync_copy(out_v, out_hbm.at[rows])


def kernel(sigs, frac_applicable_embed, bool_true_embed, bool_false_embed, frac_tf_embed):
    B, L, _ = sigs.shape
    tab = jnp.concatenate([
        frac_applicable_embed.reshape(-1),
        bool_true_embed.reshape(-1),
        bool_false_embed.reshape(-1),
        frac_tf_embed.reshape(-1),
    ])
    tab = jnp.pad(tab, (0, 64 - tab.shape[0]))
    srcr, srcl, dstc = _consts()
    idx = jnp.concatenate([srcr, srcl, dstc])
    return _sc_lookup(sigs[:, :, 0], sigs[:, :, 1], sigs[:, :, 2], sigs[:, :, 3],
                      tab, idx)


# speedup vs baseline: 43.9724x; 1.0000x over previous
import functools

import numpy as np

import jax
import jax.numpy as jnp
from jax import lax
from jax.experimental import pallas as pl
from jax.experimental.pallas import tpu as pltpu
from jax.experimental.pallas import tpu_sc as plsc

_B = 4096
_L = 50
_NW = 32
_ROWS_W = _B // _NW
_HROWS = _ROWS_W // 2
_OCTS = _HROWS // 8
_CHUNKS = (8 * _L) // 16

_BASES = (0, 22, 26, 30)
_SCALED = (True, False, False, True)


def _consts():
    jj = np.arange(8 * _L)
    srcr = (jj // _L).astype(np.int32)
    srcl = (jj % _L).astype(np.int32)
    dstc = (2 * (jj % _L)).astype(np.int32)
    return jnp.asarray(srcr), jnp.asarray(srcl), jnp.asarray(dstc)


_mesh = plsc.VectorSubcoreMesh(core_axis_name="c", subcore_axis_name="s")


@functools.partial(
    pl.kernel,
    out_type=jax.ShapeDtypeStruct((_B, _L * 8), jnp.float32),
    name="lambda_signature_lookup",
    mesh=_mesh,
    compiler_params=pltpu.CompilerParams(
        needs_layout_passes=False, use_tc_tiling_on_sc=True),
    scratch_types=[
        [pltpu.VMEM((_HROWS, _L), jnp.float32) for _ in range(4)],
        pltpu.VMEM((_HROWS, _L * 8), jnp.float32),
        pltpu.VMEM((64,), jnp.float32),
        pltpu.VMEM((3 * 8 * _L,), jnp.int32),
    ],
)
def _sc_lookup(s0_hbm, s1_hbm, s2_hbm, s3_hbm, tab_hbm, idx_hbm, out_hbm,
               sig_vs, out_v, tab_v, idx_v):
    wid = lax.axis_index("s") * 2 + lax.axis_index("c")
    base_row = wid * _ROWS_W
    pltpu.sync_copy(tab_hbm, tab_v)
    pltpu.sync_copy(idx_hbm, idx_v)
    s_hbm = (s0_hbm, s1_hbm, s2_hbm, s3_hbm)

    for h in range(2):
        rows = pl.ds(base_row + h * _HROWS, _HROWS)
        for f in range(4):
            pltpu.sync_copy(s_hbm[f].at[rows], sig_vs[f])

        for f in range(4):
            sv = sig_vs[f]
            base = _BASES[f]

            def k_body(k, _, sv=sv, base=base, scaled=_SCALED[f], fcol=f * 100):
                k16 = k * 16
                sr = idx_v[pl.ds(k16, 16)]
                sl = idx_v[pl.ds(400 + k16, 16)]
                c0 = idx_v[pl.ds(800 + k16, 16)] + fcol

                @plsc.parallel_loop(0, _OCTS, 1, unroll=8)
                def o_body(o):
                    ro = sr + o * 8
                    s = plsc.load_gather(sv, [ro, sl])
                    if scaled:
                        s = s * jnp.float32(10.0)
                    t = s.astype(jnp.int32)
                    idx = t + t + base
                    v0 = plsc.load_gather(tab_v, [idx])
                    v1 = plsc.load_gather(tab_v, [idx + 1])
                    plsc.store_scatter(out_v, [ro, c0], v0)
                    plsc.store_scatter(out_v, [ro, c0 + 1], v1)

                return 0

            lax.fori_loop(0, _CHUNKS, k_body, 0)

        pltpu.sync_copy(out_v, out_hbm.at[rows])


def kernel(sigs, frac_applicable_embed, bool_true_embed, bool_false_embed, frac_tf_embed):
    B, L, _ = sigs.shape
    tab = jnp.concatenate([
        frac_applicable_embed.reshape(-1),
        bool_true_embed.reshape(-1),
        bool_false_embed.reshape(-1),
        frac_tf_embed.reshape(-1),
    ])
    tab = jnp.pad(tab, (0, 64 - tab.shape[0]))
    srcr, srcl, dstc = _consts()
    idx = jnp.concatenate([srcr, srcl, dstc])
    return _sc_lookup(sigs[:, :, 0], sigs[:, :, 1], sigs[:, :, 2], sigs[:, :, 3],
                      tab, idx)
